# baseline (speedup 1.0000x reference)
import jax
import jax.numpy as jnp
from jax import lax
from jax.experimental import pallas as pl
from jax.experimental.pallas import tpu as pltpu
from jax.experimental.pallas import tpu_sc as plsc

B, C, DIM = 4, 96, 384
OUT = 192
NPIX = DIM * DIM
NOUT = OUT * OUT
NPLANES = B * C
NTILES = 32
PLANES_PER_TILE = NPLANES // NTILES
HALF = NPIX // 2
IDX_CHUNK = 4608
NCHUNKS = NPIX // IDX_CHUNK
WIN_PER_CHUNK = IDX_CHUNK // 4
GROUPS_PER_CHUNK = WIN_PER_CHUNK // 16

NEG = float("-inf")


def _body(x_hbm, idx_hbm, out_hbm, half_v, out_v, ib0, ib1, sem0, sem1):
    wid = lax.axis_index("s") * 2 + lax.axis_index("c")
    bufs = (ib0, ib1)
    sems = (sem0, sem1)

    def plane_body(pi, carry):
        plane = wid * PLANES_PER_TILE + pi
        for h in range(2):
            pltpu.sync_copy(x_hbm.at[plane, pl.ds(h * HALF, HALF)], half_v)
            pltpu.async_copy(idx_hbm.at[0], ib0, sem0)
            pltpu.async_copy(idx_hbm.at[1], ib1, sem1)

            def pair_body(i, carry2):
                for b in range(2):
                    c = 2 * i + b
                    buf, sem = bufs[b], sems[b]
                    pltpu.make_async_copy(idx_hbm.at[c], buf, sem).wait()

                    def group_body(g4, carry3):
                        for u in range(4):
                            acc = None
                            for j in range(4):
                                kv = buf[j, pl.ds(g4 * 64 + u * 16, 16)]
                                if h == 0:
                                    m = kv < HALF
                                    lc = jnp.minimum(kv, HALF - 1)
                                else:
                                    lidx = kv - HALF
                                    m = lidx >= 0
                                    lc = jnp.maximum(lidx, 0)
                                v = plsc.load_gather(half_v, [lc])
                                sv = jnp.where(m, v, NEG)
                                acc = sv if acc is None else jnp.maximum(acc, sv)
                            sl = pl.ds(c * WIN_PER_CHUNK + g4 * 64 + u * 16, 16)
                            if h == 0:
                                out_v[sl] = acc
                            else:
                                out_v[sl] = jnp.maximum(out_v[sl], acc)
                        return carry3

                    lax.fori_loop(0, GROUPS_PER_CHUNK // 4, group_body, 0)

                    @pl.when(c + 2 < NCHUNKS)
                    def _prefetch():
                        pltpu.async_copy(idx_hbm.at[c + 2], buf, sem)

                return carry2

            lax.fori_loop(0, NCHUNKS // 2, pair_body, 0)
        pltpu.sync_copy(out_v, out_hbm.at[plane, :])
        return carry

    lax.fori_loop(0, PLANES_PER_TILE, plane_body, 0)


def _make_kernel():
    mesh = plsc.VectorSubcoreMesh(core_axis_name="c", subcore_axis_name="s")
    return pl.kernel(
        _body,
        out_type=jax.ShapeDtypeStruct((NPLANES, NOUT), jnp.float32),
        mesh=mesh,
        scratch_types=[
            pltpu.VMEM((HALF,), jnp.float32),
            pltpu.VMEM((NOUT,), jnp.float32),
            pltpu.VMEM((4, WIN_PER_CHUNK), jnp.int32),
            pltpu.VMEM((4, WIN_PER_CHUNK), jnp.int32),
            pltpu.SemaphoreType.DMA,
            pltpu.SemaphoreType.DMA,
        ],
        compiler_params=pltpu.CompilerParams(needs_layout_passes=False),
    )


@jax.jit
def kernel(x, gather_idx):
    xf = x.reshape(NPLANES, NPIX)
    idx_r = gather_idx.reshape(NCHUNKS, WIN_PER_CHUNK, 4).transpose(0, 2, 1)
    out = _make_kernel()(xf, idx_r)
    return out.reshape(B, C, OUT, OUT)


# speedup vs baseline: 1.5080x; 1.5080x over previous
import jax
import jax.numpy as jnp
from jax import lax
from jax.experimental import pallas as pl
from jax.experimental.pallas import tpu as pltpu
from jax.experimental.pallas import tpu_sc as plsc

B, C, DIM = 4, 96, 384
OUT = 192
NPIX = DIM * DIM
NOUT = OUT * OUT
NPLANES = B * C
NTILES = 32
PLANES_PER_TILE = NPLANES // NTILES
HALF = NPIX // 2
IDX_CHUNK = 4608
NCHUNKS = NPIX // IDX_CHUNK
WIN_PER_CHUNK = IDX_CHUNK // 4
GROUPS_PER_CHUNK = WIN_PER_CHUNK // 16

NEG = float("-inf")


def _body(x_hbm, idx_hbm, out_hbm, half_v, out_v, ib0, ib1, sem0, sem1):
    wid = lax.axis_index("s") * 2 + lax.axis_index("c")
    bufs = (ib0, ib1)
    sems = (sem0, sem1)

    def plane_body(pi, carry):
        plane = wid * PLANES_PER_TILE + pi
        for h in range(2):
            pltpu.sync_copy(x_hbm.at[plane, pl.ds(h * HALF, HALF)], half_v)
            pltpu.async_copy(idx_hbm.at[0], ib0, sem0)
            pltpu.async_copy(idx_hbm.at[1], ib1, sem1)

            def pair_body(i, carry2):
                for b in range(2):
                    c = 2 * i + b
                    buf, sem = bufs[b], sems[b]
                    pltpu.make_async_copy(idx_hbm.at[c], buf, sem).wait()

                    @plsc.parallel_loop(0, GROUPS_PER_CHUNK, unroll=4)
                    def group_body(g):
                        acc = None
                        for j in range(4):
                            kv = buf[j, pl.ds(g * 16, 16)]
                            if h == 0:
                                m = kv < HALF
                                lc = jnp.minimum(kv, HALF - 1)
                            else:
                                lidx = kv - HALF
                                m = lidx >= 0
                                lc = jnp.maximum(lidx, 0)
                            v = plsc.load_gather(half_v, [lc])
                            sv = jnp.where(m, v, NEG)
                            acc = sv if acc is None else jnp.maximum(acc, sv)
                        sl = pl.ds(c * WIN_PER_CHUNK + g * 16, 16)
                        if h == 0:
                            out_v[sl] = acc
                        else:
                            out_v[sl] = jnp.maximum(out_v[sl], acc)

                    @pl.when(c + 2 < NCHUNKS)
                    def _prefetch():
                        pltpu.async_copy(idx_hbm.at[c + 2], buf, sem)

                return carry2

            lax.fori_loop(0, NCHUNKS // 2, pair_body, 0)
        pltpu.sync_copy(out_v, out_hbm.at[plane, :])
        return carry

    lax.fori_loop(0, PLANES_PER_TILE, plane_body, 0)


def _make_kernel():
    mesh = plsc.VectorSubcoreMesh(core_axis_name="c", subcore_axis_name="s")
    return pl.kernel(
        _body,
        out_type=jax.ShapeDtypeStruct((NPLANES, NOUT), jnp.float32),
        mesh=mesh,
        scratch_types=[
            pltpu.VMEM((HALF,), jnp.float32),
            pltpu.VMEM((NOUT,), jnp.float32),
            pltpu.VMEM((4, WIN_PER_CHUNK), jnp.int32),
            pltpu.VMEM((4, WIN_PER_CHUNK), jnp.int32),
            pltpu.SemaphoreType.DMA,
            pltpu.SemaphoreType.DMA,
        ],
        compiler_params=pltpu.CompilerParams(needs_layout_passes=False),
    )


@jax.jit
def kernel(x, gather_idx):
    xf = x.reshape(NPLANES, NPIX)
    idx_r = gather_idx.reshape(NCHUNKS, WIN_PER_CHUNK, 4).transpose(0, 2, 1)
    out = _make_kernel()(xf, idx_r)
    return out.reshape(B, C, OUT, OUT)


# speedup vs baseline: 1.5271x; 1.0126x over previous
import jax
import jax.numpy as jnp
from jax import lax
from jax.experimental import pallas as pl
from jax.experimental.pallas import tpu as pltpu
from jax.experimental.pallas import tpu_sc as plsc

B, C, DIM = 4, 96, 384
OUT = 192
NPIX = DIM * DIM
NOUT = OUT * OUT
NPLANES = B * C
NTILES = 32
PLANES_PER_TILE = NPLANES // NTILES
HALF = NPIX // 2
IDX_CHUNK = 4608
NCHUNKS = NPIX // IDX_CHUNK
WIN_PER_CHUNK = IDX_CHUNK // 4
GROUPS_PER_CHUNK = WIN_PER_CHUNK // 16

NEG = float("-inf")


def _body(x_hbm, idx_hbm, out_hbm, half_v, out_v, ib0, ib1, sem0, sem1):
    wid = lax.axis_index("s") * 2 + lax.axis_index("c")
    bufs = (ib0, ib1)
    sems = (sem0, sem1)
    iota4 = lax.iota(jnp.int32, 16) * 4

    def plane_body(pi, carry):
        plane = wid * PLANES_PER_TILE + pi
        for h in range(2):
            pltpu.sync_copy(x_hbm.at[plane, pl.ds(h * HALF, HALF)], half_v)
            pltpu.async_copy(idx_hbm.at[pl.ds(0, IDX_CHUNK)], ib0, sem0)
            pltpu.async_copy(idx_hbm.at[pl.ds(IDX_CHUNK, IDX_CHUNK)], ib1,
                             sem1)

            def pair_body(i, carry2):
                for b in range(2):
                    c = 2 * i + b
                    buf, sem = bufs[b], sems[b]
                    pltpu.make_async_copy(
                        idx_hbm.at[pl.ds(c * IDX_CHUNK, IDX_CHUNK)], buf,
                        sem).wait()

                    @plsc.parallel_loop(0, GROUPS_PER_CHUNK, unroll=4)
                    def group_body(g):
                        base = g * 64 + iota4
                        acc = None
                        for j in range(4):
                            kv = plsc.load_gather(buf, [base + j])
                            if h == 0:
                                m = kv < HALF
                                lc = jnp.minimum(kv, HALF - 1)
                            else:
                                lidx = kv - HALF
                                m = lidx >= 0
                                lc = jnp.maximum(lidx, 0)
                            v = plsc.load_gather(half_v, [lc])
                            sv = jnp.where(m, v, NEG)
                            acc = sv if acc is None else jnp.maximum(acc, sv)
                        sl = pl.ds(c * WIN_PER_CHUNK + g * 16, 16)
                        if h == 0:
                            out_v[sl] = acc
                        else:
                            out_v[sl] = jnp.maximum(out_v[sl], acc)

                    @pl.when(c + 2 < NCHUNKS)
                    def _prefetch():
                        pltpu.async_copy(
                            idx_hbm.at[pl.ds((c + 2) * IDX_CHUNK, IDX_CHUNK)],
                            buf, sem)

                return carry2

            lax.fori_loop(0, NCHUNKS // 2, pair_body, 0)
        pltpu.sync_copy(out_v, out_hbm.at[plane, :])
        return carry

    lax.fori_loop(0, PLANES_PER_TILE, plane_body, 0)


def _make_kernel():
    mesh = plsc.VectorSubcoreMesh(core_axis_name="c", subcore_axis_name="s")
    return pl.kernel(
        _body,
        out_type=jax.ShapeDtypeStruct((NPLANES, NOUT), jnp.float32),
        mesh=mesh,
        scratch_types=[
            pltpu.VMEM((HALF,), jnp.float32),
            pltpu.VMEM((NOUT,), jnp.float32),
            pltpu.VMEM((IDX_CHUNK,), jnp.int32),
            pltpu.VMEM((IDX_CHUNK,), jnp.int32),
            pltpu.SemaphoreType.DMA,
            pltpu.SemaphoreType.DMA,
        ],
        compiler_params=pltpu.CompilerParams(needs_layout_passes=False),
    )


@jax.jit
def kernel(x, gather_idx):
    xf = x.reshape(NPLANES, NPIX)
    idx_flat = gather_idx.reshape(-1)
    out = _make_kernel()(xf, idx_flat)
    return out.reshape(B, C, OUT, OUT)


# speedup vs baseline: 1.5872x; 1.0394x over previous
import jax
import jax.numpy as jnp
from jax import lax
from jax.experimental import pallas as pl
from jax.experimental.pallas import tpu as pltpu
from jax.experimental.pallas import tpu_sc as plsc

B, C, DIM = 4, 96, 384
OUT = 192
NPIX = DIM * DIM
NOUT = OUT * OUT
NPLANES = B * C
NTILES = 32
PLANES_PER_TILE = NPLANES // NTILES
HALF = NPIX // 2
IDX_CHUNK = 4608
NCHUNKS = NPIX // IDX_CHUNK
WIN_PER_CHUNK = IDX_CHUNK // 4
GROUPS_PER_CHUNK = WIN_PER_CHUNK // 16

NEG = float("-inf")


def _body(x_hbm, idx_hbm, out_hbm, half_v, out_v, ib0, ib1, tidx_s,
          sem0, sem1):
    sid = lax.axis_index("s")
    wid = sid * 2 + lax.axis_index("c")
    bufs = (ib0, ib1)
    sems = (sem0, sem1)
    iota4 = lax.iota(jnp.int32, 16) * 4

    for t in range(2):
        c0 = sid * 2 + t
        pltpu.sync_copy(idx_hbm.at[pl.ds(c0 * IDX_CHUNK, IDX_CHUNK)], ib0)

        for j in range(4):
            @plsc.parallel_loop(0, GROUPS_PER_CHUNK, unroll=4)
            def tr_body(g):
                v = plsc.load_gather(ib0, [g * 64 + iota4 + j])
                ib1[pl.ds(j * WIN_PER_CHUNK + g * 16, 16)] = v

        pltpu.sync_copy(ib1, tidx_s.at[pl.ds(c0 * IDX_CHUNK, IDX_CHUNK)])
    plsc.subcore_barrier()

    def plane_body(pi, carry):
        plane = wid * PLANES_PER_TILE + pi
        for h in range(2):
            pltpu.sync_copy(x_hbm.at[plane, pl.ds(h * HALF, HALF)], half_v)
            pltpu.async_copy(tidx_s.at[pl.ds(0, IDX_CHUNK)], ib0, sem0)
            pltpu.async_copy(tidx_s.at[pl.ds(IDX_CHUNK, IDX_CHUNK)], ib1,
                             sem1)

            def pair_body(i, carry2):
                for b in range(2):
                    c = 2 * i + b
                    buf, sem = bufs[b], sems[b]
                    pltpu.make_async_copy(
                        tidx_s.at[pl.ds(c * IDX_CHUNK, IDX_CHUNK)], buf,
                        sem).wait()

                    @plsc.parallel_loop(0, GROUPS_PER_CHUNK, unroll=4)
                    def group_body(g):
                        acc = None
                        for j in range(4):
                            kv = buf[pl.ds(j * WIN_PER_CHUNK + g * 16, 16)]
                            if h == 0:
                                m = kv < HALF
                                lc = jnp.minimum(kv, HALF - 1)
                            else:
                                lidx = kv - HALF
                                m = lidx >= 0
                                lc = jnp.maximum(lidx, 0)
                            v = plsc.load_gather(half_v, [lc])
                            sv = jnp.where(m, v, NEG)
                            acc = sv if acc is None else jnp.maximum(acc, sv)
                        sl = pl.ds(c * WIN_PER_CHUNK + g * 16, 16)
                        if h == 0:
                            out_v[sl] = acc
                        else:
                            out_v[sl] = jnp.maximum(out_v[sl], acc)

                    @pl.when(c + 2 < NCHUNKS)
                    def _prefetch():
                        pltpu.async_copy(
                            tidx_s.at[pl.ds((c + 2) * IDX_CHUNK, IDX_CHUNK)],
                            buf, sem)

                return carry2

            lax.fori_loop(0, NCHUNKS // 2, pair_body, 0)
        pltpu.sync_copy(out_v, out_hbm.at[plane, :])
        return carry

    lax.fori_loop(0, PLANES_PER_TILE, plane_body, 0)


def _make_kernel():
    mesh = plsc.VectorSubcoreMesh(core_axis_name="c", subcore_axis_name="s")
    return pl.kernel(
        _body,
        out_type=jax.ShapeDtypeStruct((NPLANES, NOUT), jnp.float32),
        mesh=mesh,
        scratch_types=[
            pltpu.VMEM((HALF,), jnp.float32),
            pltpu.VMEM((NOUT,), jnp.float32),
            pltpu.VMEM((IDX_CHUNK,), jnp.int32),
            pltpu.VMEM((IDX_CHUNK,), jnp.int32),
            pltpu.VMEM_SHARED((NPIX,), jnp.int32),
            pltpu.SemaphoreType.DMA,
            pltpu.SemaphoreType.DMA,
        ],
        compiler_params=pltpu.CompilerParams(needs_layout_passes=False),
    )


@jax.jit
def kernel(x, gather_idx):
    xf = x.reshape(NPLANES, NPIX)
    idx_flat = gather_idx.reshape(-1)
    out = _make_kernel()(xf, idx_flat)
    return out.reshape(B, C, OUT, OUT)
